# Initial kernel scaffold; baseline (speedup 1.0000x reference)
import jax
import jax.numpy as jnp
from jax.experimental import pallas as pl


def kernel(batch, triples, weights0, bias0, weights1, bias1, relations):
    raise NotImplementedError("write your pallas kernel here")


# speedup vs baseline: 17.6423x; 17.6423x over previous
import functools

import jax
import jax.numpy as jnp
from jax import lax
from jax.experimental import pallas as pl
from jax.experimental.pallas import tpu as pltpu
from jax.experimental.pallas import tpu_sc as plsc

NC = 2
NS = 16
NW = NC * NS
LANES = 16

CH = 1024
ZROWS = 800


def _sc_mesh():
  return plsc.VectorSubcoreMesh(
      core_axis_name="c", subcore_axis_name="s", num_cores=NC,
      num_subcores=NS)


_SC_PARAMS = pltpu.CompilerParams(
    use_tc_tiling_on_sc=False, needs_layout_passes=False)


def _scatter_phase(table, idx2d, dst2d, npad, nume, with_deg):
  ept = nume // NW
  nch = ept // CH
  rpt = npad // NS

  out_type = [jax.ShapeDtypeStruct((NC, npad, 16), jnp.float32)]
  if with_deg:
    out_type.append(jax.ShapeDtypeStruct((NC * npad,), jnp.float32))

  scratch = [
      pltpu.VMEM_SHARED((npad, 16), jnp.float32),
      pltpu.VMEM((ZROWS, 16), jnp.float32),
      pltpu.VMEM((8, 128), jnp.int32),
      pltpu.VMEM((8, 128), jnp.int32),
      pltpu.VMEM((CH, 16), jnp.float32),
      pltpu.SemaphoreType.DMA,
  ]
  if with_deg:
    scratch += [
        pltpu.VMEM_SHARED((npad,), jnp.float32),
        pltpu.VMEM((rpt,), jnp.float32),
        pltpu.VMEM((128,), jnp.float32),
    ]

  @functools.partial(
      pl.kernel, out_type=tuple(out_type), mesh=_sc_mesh(),
      scratch_types=tuple(scratch), compiler_params=_SC_PARAMS)
  def k(table_h, idx_h, dst_h, *rest):
    if with_deg:
      (hp_h, dp_h, hacc, z, idxb, dstb, rows, sem, dacc, zd, ones) = rest
    else:
      (hp_h, hacc, z, idxb, dstb, rows, sem) = rest
    cid = lax.axis_index("c")
    sid = lax.axis_index("s")
    wid = sid * NC + cid

    zero16 = jnp.zeros((16,), jnp.float32)

    def zbody(i, carry):
      z[i] = zero16
      return carry
    lax.fori_loop(0, ZROWS, zbody, 0)
    if with_deg:
      def zdbody(i, carry):
        zd[pl.ds(i * 16, 16)] = zero16
        return carry
      lax.fori_loop(0, rpt // 16, zdbody, 0)
      one16 = jnp.ones((16,), jnp.float32)
      for i in range(8):
        ones[pl.ds(i * 16, 16)] = one16

    for j in range(rpt // ZROWS):
      pltpu.sync_copy(z, hacc.at[pl.ds(sid * rpt + j * ZROWS, ZROWS)])
    if with_deg:
      pltpu.sync_copy(zd, dacc.at[pl.ds(sid * rpt, rpt)])
    plsc.subcore_barrier()

    def chunk(c, carry):
      row0 = wid * (ept // 128) + c * (CH // 128)
      pltpu.sync_copy(idx_h.at[pl.ds(row0, 8)], idxb)
      pltpu.sync_copy(dst_h.at[pl.ds(row0, 8)], dstb)
      descs = []
      for j in range(8):
        descs.append(pltpu.async_copy(
            table_h.at[idxb.at[j]], rows.at[pl.ds(j * 128, 128)], sem))
      for d in descs:
        d.wait()
      for j in range(8):
        pltpu.sync_copy(rows.at[pl.ds(j * 128, 128)],
                        hacc.at[dstb.at[j]], add=True)
        if with_deg:
          pltpu.sync_copy(ones, dacc.at[dstb.at[j]], add=True)
      return carry
    lax.fori_loop(0, nch, chunk, 0)

    plsc.subcore_barrier()
    pltpu.sync_copy(hacc.at[pl.ds(sid * rpt, rpt)],
                    hp_h.at[cid, pl.ds(sid * rpt, rpt)])
    if with_deg:
      pltpu.sync_copy(dacc.at[pl.ds(sid * rpt, rpt)],
                      dp_h.at[pl.ds(cid * npad + sid * rpt, rpt)])

  return k(table, idx2d, dst2d)


def _dense_phase(hp0, hp1, dpt, w0self, bias0, w1cat, npad, rtot):
  blk = ZROWS
  grid = npad // blk
  wcols = rtot * 16

  def body(hp0_r, hp1_r, dp_r, w0s_r, b0_r, w1_r, nw_r, ivb_r, nws_r):
    deg = dp_r[:, 0:1] + dp_r[:, 1:2] + 1.0
    invd = 1.0 / deg
    h = jnp.maximum(
        (hp0_r[...] + hp1_r[...] + w0s_r[...]) * invd + b0_r[...], 0.0)
    nw = jnp.dot(h, w1_r[...], preferred_element_type=jnp.float32)
    nw_r[...] = nw
    ivb_r[...] = jnp.broadcast_to(invd, (blk, 16))
    nws_r[...] = nw[:, wcols - 16:]

  return pl.pallas_call(
      body,
      grid=(grid,),
      in_specs=[
          pl.BlockSpec((blk, 16), lambda i: (i, 0)),
          pl.BlockSpec((blk, 16), lambda i: (i, 0)),
          pl.BlockSpec((blk, 2), lambda i: (i, 0)),
          pl.BlockSpec((blk, 16), lambda i: (i, 0)),
          pl.BlockSpec((1, 16), lambda i: (0, 0)),
          pl.BlockSpec((16, wcols), lambda i: (0, 0)),
      ],
      out_specs=[
          pl.BlockSpec((blk, wcols), lambda i: (i, 0)),
          pl.BlockSpec((blk, 16), lambda i: (i, 0)),
          pl.BlockSpec((blk, 16), lambda i: (i, 0)),
      ],
      out_shape=[
          jax.ShapeDtypeStruct((npad, wcols), jnp.float32),
          jax.ShapeDtypeStruct((npad, 16), jnp.float32),
          jax.ShapeDtypeStruct((npad, 16), jnp.float32),
      ],
  )(hp0, hp1, dpt, w0self, bias0, w1cat)


def _normalize_phase(c0, c1, nws, ivb, bias1, npad):
  rpt = npad // NW
  nchunk = rpt // ZROWS

  @functools.partial(
      pl.kernel,
      out_type=jax.ShapeDtypeStruct((npad, 16), jnp.float32),
      mesh=_sc_mesh(),
      scratch_types=(
          pltpu.VMEM((ZROWS, 16), jnp.float32),
          pltpu.VMEM((ZROWS, 16), jnp.float32),
          pltpu.VMEM((ZROWS, 16), jnp.float32),
          pltpu.VMEM((ZROWS, 16), jnp.float32),
          pltpu.VMEM((ZROWS, 16), jnp.float32),
          pltpu.VMEM((16,), jnp.float32),
      ),
      compiler_params=_SC_PARAMS)
  def k(c0_h, c1_h, nws_h, ivb_h, b1_h, h2_h, c0b, c1b, nwb, ivbb, outb,
        b1v):
    cid = lax.axis_index("c")
    sid = lax.axis_index("s")
    wid = sid * NC + cid
    base = wid * rpt
    pltpu.sync_copy(b1_h, b1v)
    b1 = b1v[...]
    for j in range(nchunk):
      r0 = base + j * ZROWS
      pltpu.sync_copy(c0_h.at[pl.ds(r0, ZROWS)], c0b)
      pltpu.sync_copy(c1_h.at[pl.ds(r0, ZROWS)], c1b)
      pltpu.sync_copy(nws_h.at[pl.ds(r0, ZROWS)], nwb)
      pltpu.sync_copy(ivb_h.at[pl.ds(r0, ZROWS)], ivbb)

      def body(r, carry):
        outb[r] = (c0b[r] + c1b[r] + nwb[r]) * ivbb[r] + b1
        return carry
      lax.fori_loop(0, ZROWS, body, 0)
      pltpu.sync_copy(outb, h2_h.at[pl.ds(r0, ZROWS)])

  return k(c0, c1, nws, ivb, bias1)


def _decoder_phase(h2, qs, qp, qo, relations, nb):
  qpt = nb // NW

  @functools.partial(
      pl.kernel,
      out_type=jax.ShapeDtypeStruct((nb,), jnp.float32),
      mesh=_sc_mesh(),
      scratch_types=(
          pltpu.VMEM((qpt // 128, 128), jnp.int32),
          pltpu.VMEM((qpt // 128, 128), jnp.int32),
          pltpu.VMEM((qpt // 128, 128), jnp.int32),
          pltpu.VMEM((qpt, 16), jnp.float32),
          pltpu.VMEM((qpt, 16), jnp.float32),
          pltpu.VMEM((qpt, 16), jnp.float32),
          pltpu.VMEM((qpt,), jnp.float32),
          pltpu.SemaphoreType.DMA,
      ),
      compiler_params=_SC_PARAMS)
  def k(h2_h, qs_h, qp_h, qo_h, rel_h, sc_h, qsb, qpb, qob, hs, hr, ho,
        scb, sem):
    cid = lax.axis_index("c")
    sid = lax.axis_index("s")
    wid = sid * NC + cid
    qrow = wid * (qpt // 128)
    pltpu.sync_copy(qs_h.at[pl.ds(qrow, qpt // 128)], qsb)
    pltpu.sync_copy(qp_h.at[pl.ds(qrow, qpt // 128)], qpb)
    pltpu.sync_copy(qo_h.at[pl.ds(qrow, qpt // 128)], qob)
    descs = []
    for j in range(qpt // 128):
      descs.append(pltpu.async_copy(
          h2_h.at[qsb.at[j]], hs.at[pl.ds(j * 128, 128)], sem))
      descs.append(pltpu.async_copy(
          rel_h.at[qpb.at[j]], hr.at[pl.ds(j * 128, 128)], sem))
      descs.append(pltpu.async_copy(
          h2_h.at[qob.at[j]], ho.at[pl.ds(j * 128, 128)], sem))
    for d in descs:
      d.wait()
    lane = lax.iota(jnp.int32, 16)
    acc = jnp.zeros((16,), jnp.float32)
    for q in range(qpt):
      v = hs[q] * hr[q] * ho[q]
      s = jnp.sum(v)
      acc = jnp.where(lane == (q % 16), s, acc)
      if q % 16 == 15:
        scb[pl.ds((q // 16) * 16, 16)] = acc
    pltpu.sync_copy(scb, sc_h.at[pl.ds(wid * qpt, qpt)])

  return k(h2, qs, qp, qo, relations)


def kernel(batch, triples, weights0, bias0, weights1, bias1, relations):
  rtot, n, hid = weights0.shape
  r = relations.shape[0]
  t = triples.shape[0]
  nb = batch.shape[0]
  assert hid == 16 and weights1.shape[2] == 16

  npad = -(-n // 2048) * 2048
  if npad == n:
    npad += 2048
  tpad = -(-t // (NW * CH)) * (NW * CH)

  s = triples[:, 0]
  p = triples[:, 1]
  o = triples[:, 2]

  idx0 = jnp.concatenate([p * n + s, (p + r) * n + o])
  idx1 = jnp.concatenate([s * rtot + p, o * rtot + (p + r)])
  dst = jnp.concatenate([o, s])

  e = 2 * t
  epad = 2 * tpad
  pe = epad - e
  idx0 = jnp.concatenate([idx0, jnp.zeros((pe,), jnp.int32)])
  idx1 = jnp.concatenate([idx1, jnp.zeros((pe,), jnp.int32)])
  dst = jnp.concatenate([dst, jnp.full((pe,), npad - 1, jnp.int32)])
  idx0 = idx0.reshape(epad // 128, 128)
  idx1 = idx1.reshape(epad // 128, 128)
  dst = dst.reshape(epad // 128, 128)

  w0flat = weights0.reshape(rtot * n, 16)
  hp, dp = _scatter_phase(w0flat, idx0, dst, npad, epad, with_deg=True)

  w0self = jnp.pad(weights0[2 * r], ((0, npad - n), (0, 0)))
  w1cat = weights1.transpose(1, 0, 2).reshape(16, rtot * 16)
  dpt = jnp.stack([dp[:npad], dp[npad:]], axis=1)
  nw, ivb, nws = _dense_phase(hp[0], hp[1], dpt, w0self,
                              bias0.reshape(1, 16), w1cat, npad, rtot)

  nwflat = nw.reshape(npad * rtot, 16)
  (cp,) = _scatter_phase(nwflat, idx1, dst, npad, epad, with_deg=False)

  h2 = _normalize_phase(cp[0], cp[1], nws, ivb, bias1, npad)

  qs = batch[:, 0].reshape(nb // 128, 128)
  qp = batch[:, 1].reshape(nb // 128, 128)
  qo = batch[:, 2].reshape(nb // 128, 128)
  scores = _decoder_phase(h2, qs, qp, qo, relations, nb)
  return scores


# speedup vs baseline: 19.1926x; 1.0879x over previous
import functools

import jax
import jax.numpy as jnp
from jax import lax
from jax.experimental import pallas as pl
from jax.experimental.pallas import tpu as pltpu
from jax.experimental.pallas import tpu_sc as plsc

NC = 2
NS = 16
NW = NC * NS
LANES = 16

CH = 1024
ZROWS = 800


def _sc_mesh():
  return plsc.VectorSubcoreMesh(
      core_axis_name="c", subcore_axis_name="s", num_cores=NC,
      num_subcores=NS)


_SC_PARAMS = pltpu.CompilerParams(
    use_tc_tiling_on_sc=False, needs_layout_passes=False)


def _scatter_phase(table, idx2d, dst2d, npad, nume, with_deg):
  ept = nume // NW
  nch = ept // CH
  rpt = npad // NS

  out_type = [jax.ShapeDtypeStruct((NC, npad, 16), jnp.float32)]
  if with_deg:
    out_type.append(jax.ShapeDtypeStruct((NC * npad,), jnp.float32))

  scratch = [
      pltpu.VMEM_SHARED((npad, 16), jnp.float32),
      pltpu.VMEM((ZROWS, 16), jnp.float32),
      pltpu.VMEM((8, 128), jnp.int32),
      pltpu.VMEM((8, 128), jnp.int32),
      pltpu.VMEM((8, 128), jnp.int32),
      pltpu.VMEM((8, 128), jnp.int32),
      pltpu.VMEM((CH, 16), jnp.float32),
      pltpu.VMEM((CH, 16), jnp.float32),
      pltpu.SemaphoreType.DMA,
  ]
  if with_deg:
    scratch += [
        pltpu.VMEM_SHARED((npad,), jnp.float32),
        pltpu.VMEM((rpt,), jnp.float32),
        pltpu.VMEM((128,), jnp.float32),
    ]

  @functools.partial(
      pl.kernel, out_type=tuple(out_type), mesh=_sc_mesh(),
      scratch_types=tuple(scratch), compiler_params=_SC_PARAMS)
  def k(table_h, idx_h, dst_h, *rest):
    if with_deg:
      (hp_h, dp_h, hacc, z, idxb0, idxb1, dstb0, dstb1, rows0, rows1,
       sem, dacc, zd, ones) = rest
    else:
      (hp_h, hacc, z, idxb0, idxb1, dstb0, dstb1, rows0, rows1,
       sem) = rest
    cid = lax.axis_index("c")
    sid = lax.axis_index("s")
    wid = sid * NC + cid

    zero16 = jnp.zeros((16,), jnp.float32)

    def zbody(i, carry):
      z[i] = zero16
      return carry
    lax.fori_loop(0, ZROWS, zbody, 0)
    if with_deg:
      def zdbody(i, carry):
        zd[pl.ds(i * 16, 16)] = zero16
        return carry
      lax.fori_loop(0, rpt // 16, zdbody, 0)
      one16 = jnp.ones((16,), jnp.float32)
      for i in range(8):
        ones[pl.ds(i * 16, 16)] = one16

    for j in range(rpt // ZROWS):
      pltpu.sync_copy(z, hacc.at[pl.ds(sid * rpt + j * ZROWS, ZROWS)])
    if with_deg:
      pltpu.sync_copy(zd, dacc.at[pl.ds(sid * rpt, rpt)])
    plsc.subcore_barrier()

    bufs = ((idxb0, dstb0, rows0), (idxb1, dstb1, rows1))

    def load_chunk(c, ib, db):
      row0 = wid * (ept // 128) + c * (CH // 128)
      pltpu.sync_copy(idx_h.at[pl.ds(row0, 8)], ib)
      pltpu.sync_copy(dst_h.at[pl.ds(row0, 8)], db)

    def fire_gathers(ib, rb):
      for j in range(8):
        pltpu.async_copy(
            table_h.at[ib.at[j]], rb.at[pl.ds(j * 128, 128)], sem)

    def wait_gathers(ib, rb):
      for j in range(8):
        pltpu.make_async_copy(
            table_h.at[ib.at[j]], rb.at[pl.ds(j * 128, 128)], sem).wait()

    def scatter(db, rb):
      for j in range(8):
        pltpu.sync_copy(rb.at[pl.ds(j * 128, 128)],
                        hacc.at[db.at[j]], add=True)
        if with_deg:
          pltpu.sync_copy(ones, dacc.at[db.at[j]], add=True)

    load_chunk(0, idxb0, dstb0)
    fire_gathers(idxb0, rows0)

    def pair(i, carry):
      for b in range(2):
        c = 2 * i + b
        ib, db, rb = bufs[b]
        ib2, db2, rb2 = bufs[1 - b]
        wait_gathers(ib, rb)

        @pl.when(c + 1 < nch)
        def _():
          load_chunk(c + 1, ib2, db2)
          fire_gathers(ib2, rb2)

        scatter(db, rb)
      return carry
    lax.fori_loop(0, nch // 2, pair, 0)

    plsc.subcore_barrier()
    pltpu.sync_copy(hacc.at[pl.ds(sid * rpt, rpt)],
                    hp_h.at[cid, pl.ds(sid * rpt, rpt)])
    if with_deg:
      pltpu.sync_copy(dacc.at[pl.ds(sid * rpt, rpt)],
                      dp_h.at[pl.ds(cid * npad + sid * rpt, rpt)])

  return k(table, idx2d, dst2d)


def _dense_phase(hp0, hp1, dpt, w0self, bias0, w1cat, npad, rtot):
  blk = ZROWS
  grid = npad // blk
  wcols = rtot * 16

  def body(hp0_r, hp1_r, dp_r, w0s_r, b0_r, w1_r, nw_r, ivb_r, nws_r):
    deg = dp_r[:, 0:1] + dp_r[:, 1:2] + 1.0
    invd = 1.0 / deg
    h = jnp.maximum(
        (hp0_r[...] + hp1_r[...] + w0s_r[...]) * invd + b0_r[...], 0.0)
    nw = jnp.dot(h, w1_r[...], preferred_element_type=jnp.float32)
    nw_r[...] = nw
    ivb_r[...] = jnp.broadcast_to(invd, (blk, 16))
    nws_r[...] = nw[:, wcols - 16:]

  return pl.pallas_call(
      body,
      grid=(grid,),
      in_specs=[
          pl.BlockSpec((blk, 16), lambda i: (i, 0)),
          pl.BlockSpec((blk, 16), lambda i: (i, 0)),
          pl.BlockSpec((blk, 2), lambda i: (i, 0)),
          pl.BlockSpec((blk, 16), lambda i: (i, 0)),
          pl.BlockSpec((1, 16), lambda i: (0, 0)),
          pl.BlockSpec((16, wcols), lambda i: (0, 0)),
      ],
      out_specs=[
          pl.BlockSpec((blk, wcols), lambda i: (i, 0)),
          pl.BlockSpec((blk, 16), lambda i: (i, 0)),
          pl.BlockSpec((blk, 16), lambda i: (i, 0)),
      ],
      out_shape=[
          jax.ShapeDtypeStruct((npad, wcols), jnp.float32),
          jax.ShapeDtypeStruct((npad, 16), jnp.float32),
          jax.ShapeDtypeStruct((npad, 16), jnp.float32),
      ],
  )(hp0, hp1, dpt, w0self, bias0, w1cat)


def _normalize_phase(c0, c1, nws, ivb, bias1, npad):
  rpt = npad // NW
  nchunk = rpt // ZROWS

  @functools.partial(
      pl.kernel,
      out_type=jax.ShapeDtypeStruct((npad, 16), jnp.float32),
      mesh=_sc_mesh(),
      scratch_types=(
          pltpu.VMEM((ZROWS, 16), jnp.float32),
          pltpu.VMEM((ZROWS, 16), jnp.float32),
          pltpu.VMEM((ZROWS, 16), jnp.float32),
          pltpu.VMEM((ZROWS, 16), jnp.float32),
          pltpu.VMEM((ZROWS, 16), jnp.float32),
          pltpu.VMEM((16,), jnp.float32),
      ),
      compiler_params=_SC_PARAMS)
  def k(c0_h, c1_h, nws_h, ivb_h, b1_h, h2_h, c0b, c1b, nwb, ivbb, outb,
        b1v):
    cid = lax.axis_index("c")
    sid = lax.axis_index("s")
    wid = sid * NC + cid
    base = wid * rpt
    pltpu.sync_copy(b1_h, b1v)
    b1 = b1v[...]
    for j in range(nchunk):
      r0 = base + j * ZROWS
      pltpu.sync_copy(c0_h.at[pl.ds(r0, ZROWS)], c0b)
      pltpu.sync_copy(c1_h.at[pl.ds(r0, ZROWS)], c1b)
      pltpu.sync_copy(nws_h.at[pl.ds(r0, ZROWS)], nwb)
      pltpu.sync_copy(ivb_h.at[pl.ds(r0, ZROWS)], ivbb)

      def body(r, carry):
        outb[r] = (c0b[r] + c1b[r] + nwb[r]) * ivbb[r] + b1
        return carry
      lax.fori_loop(0, ZROWS, body, 0)
      pltpu.sync_copy(outb, h2_h.at[pl.ds(r0, ZROWS)])

  return k(c0, c1, nws, ivb, bias1)


def _decoder_phase(h2, qs, qp, qo, relations, nb):
  qpt = nb // NW

  @functools.partial(
      pl.kernel,
      out_type=jax.ShapeDtypeStruct((nb,), jnp.float32),
      mesh=_sc_mesh(),
      scratch_types=(
          pltpu.VMEM((qpt // 128, 128), jnp.int32),
          pltpu.VMEM((qpt // 128, 128), jnp.int32),
          pltpu.VMEM((qpt // 128, 128), jnp.int32),
          pltpu.VMEM((qpt, 16), jnp.float32),
          pltpu.VMEM((qpt, 16), jnp.float32),
          pltpu.VMEM((qpt, 16), jnp.float32),
          pltpu.VMEM((qpt,), jnp.float32),
          pltpu.SemaphoreType.DMA,
      ),
      compiler_params=_SC_PARAMS)
  def k(h2_h, qs_h, qp_h, qo_h, rel_h, sc_h, qsb, qpb, qob, hs, hr, ho,
        scb, sem):
    cid = lax.axis_index("c")
    sid = lax.axis_index("s")
    wid = sid * NC + cid
    qrow = wid * (qpt // 128)
    pltpu.sync_copy(qs_h.at[pl.ds(qrow, qpt // 128)], qsb)
    pltpu.sync_copy(qp_h.at[pl.ds(qrow, qpt // 128)], qpb)
    pltpu.sync_copy(qo_h.at[pl.ds(qrow, qpt // 128)], qob)
    descs = []
    for j in range(qpt // 128):
      descs.append(pltpu.async_copy(
          h2_h.at[qsb.at[j]], hs.at[pl.ds(j * 128, 128)], sem))
      descs.append(pltpu.async_copy(
          rel_h.at[qpb.at[j]], hr.at[pl.ds(j * 128, 128)], sem))
      descs.append(pltpu.async_copy(
          h2_h.at[qob.at[j]], ho.at[pl.ds(j * 128, 128)], sem))
    for d in descs:
      d.wait()
    lane = lax.iota(jnp.int32, 16)
    acc = jnp.zeros((16,), jnp.float32)
    for q in range(qpt):
      v = hs[q] * hr[q] * ho[q]
      s = jnp.sum(v)
      acc = jnp.where(lane == (q % 16), s, acc)
      if q % 16 == 15:
        scb[pl.ds((q // 16) * 16, 16)] = acc
    pltpu.sync_copy(scb, sc_h.at[pl.ds(wid * qpt, qpt)])

  return k(h2, qs, qp, qo, relations)


def kernel(batch, triples, weights0, bias0, weights1, bias1, relations):
  rtot, n, hid = weights0.shape
  r = relations.shape[0]
  t = triples.shape[0]
  nb = batch.shape[0]
  assert hid == 16 and weights1.shape[2] == 16

  npad = -(-n // 2048) * 2048
  if npad == n:
    npad += 2048
  tpad = -(-t // (NW * CH)) * (NW * CH)

  s = triples[:, 0]
  p = triples[:, 1]
  o = triples[:, 2]

  idx0 = jnp.concatenate([p * n + s, (p + r) * n + o])
  idx1 = jnp.concatenate([s * rtot + p, o * rtot + (p + r)])
  dst = jnp.concatenate([o, s])

  e = 2 * t
  epad = 2 * tpad
  pe = epad - e
  idx0 = jnp.concatenate([idx0, jnp.zeros((pe,), jnp.int32)])
  idx1 = jnp.concatenate([idx1, jnp.zeros((pe,), jnp.int32)])
  dst = jnp.concatenate([dst, jnp.full((pe,), npad - 1, jnp.int32)])
  idx0 = idx0.reshape(epad // 128, 128)
  idx1 = idx1.reshape(epad // 128, 128)
  dst = dst.reshape(epad // 128, 128)

  w0flat = weights0.reshape(rtot * n, 16)
  hp, dp = _scatter_phase(w0flat, idx0, dst, npad, epad, with_deg=True)

  w0self = jnp.pad(weights0[2 * r], ((0, npad - n), (0, 0)))
  w1cat = weights1.transpose(1, 0, 2).reshape(16, rtot * 16)
  dpt = jnp.stack([dp[:npad], dp[npad:]], axis=1)
  nw, ivb, nws = _dense_phase(hp[0], hp[1], dpt, w0self,
                              bias0.reshape(1, 16), w1cat, npad, rtot)

  nwflat = nw.reshape(npad * rtot, 16)
  (cp,) = _scatter_phase(nwflat, idx1, dst, npad, epad, with_deg=False)

  h2 = _normalize_phase(cp[0], cp[1], nws, ivb, bias1, npad)

  qs = batch[:, 0].reshape(nb // 128, 128)
  qp = batch[:, 1].reshape(nb // 128, 128)
  qo = batch[:, 2].reshape(nb // 128, 128)
  scores = _decoder_phase(h2, qs, qp, qo, relations, nb)
  return scores
